# baseline (speedup 1.0000x reference)
import functools
import math

import jax
import jax.numpy as jnp
from jax import lax
from jax.experimental import pallas as pl
from jax.experimental.pallas import tpu as pltpu
from jax.experimental.pallas import tpu_sc as plsc

_NC = 2
_NS = 16
_NW = _NC * _NS
_LANES = 16

_EMB = 64
_SCALE = math.sqrt(_EMB)

_NBUF = 4
_IDX_CHUNKS = (120, 80)


def _body(tok_hbm, table_hbm, out_hbm, idx_v, rows, gsems, osems,
          *, rows_per_w, seq, n_steps):
    wid = lax.axis_index("s") * _NC + lax.axis_index("c")
    row0 = wid * rows_per_w

    pltpu.sync_copy(tok_hbm.at[pl.ds(row0, rows_per_w)], idx_v)

    def gathers(b, r):
        cps, off = [], 0
        for w in _IDX_CHUNKS:
            cps.append(pltpu.make_async_copy(
                table_hbm.at[idx_v.at[r, pl.ds(off, w)]],
                rows.at[b, pl.ds(off, w)],
                gsems[b]))
            off += w
        return cps

    def out_copy(b, r):
        return pltpu.make_async_copy(rows.at[b], out_hbm.at[row0 + r], osems[b])

    def scale(b):
        @plsc.parallel_loop(0, seq, 1, unroll=4)
        def _(r):
            for c in range(_EMB // _LANES):
                sl = pl.ds(c * _LANES, _LANES)
                rows[b, r, sl] = rows[b, r, sl] * _SCALE

    for b in range(_NBUF):
        for cp in gathers(b, b):
            cp.start()

    @pl.loop(0, n_steps)
    def _(s):
        rb = s * _NBUF
        for b in range(_NBUF):
            @pl.when(s > 0)
            def _():
                out_copy(b, rb - _NBUF + b).wait()
                for cp in gathers(b, rb + b):
                    cp.start()
        for b in range(_NBUF):
            for cp in gathers(b, rb + b):
                cp.wait()
            scale(b)
            out_copy(b, rb + b).start()

    for b in range(_NBUF):
        out_copy(b, (n_steps - 1) * _NBUF + b).wait()


def kernel(tokens, table):
    bsz, seq = tokens.shape
    vocab, emb = table.shape
    assert emb == _EMB and seq == sum(_IDX_CHUNKS) and bsz % (_NW * _NBUF) == 0
    rows_per_w = bsz // _NW
    n_steps = rows_per_w // _NBUF

    tokens = tokens.astype(jnp.int32)
    table = table.astype(jnp.float32)

    mesh = plsc.VectorSubcoreMesh(
        core_axis_name="c", subcore_axis_name="s",
        num_cores=_NC, num_subcores=_NS)

    body = functools.partial(_body, rows_per_w=rows_per_w, seq=seq,
                             n_steps=n_steps)
    return pl.kernel(
        body,
        out_type=jax.ShapeDtypeStruct((bsz, seq, _EMB), jnp.float32),
        mesh=mesh,
        compiler_params=pltpu.CompilerParams(use_tc_tiling_on_sc=False),
        scratch_types=dict(
            idx_v=pltpu.VMEM((rows_per_w, seq), jnp.int32),
            rows=pltpu.VMEM((_NBUF, seq, _EMB), jnp.float32),
            gsems=[pltpu.SemaphoreType.DMA] * _NBUF,
            osems=[pltpu.SemaphoreType.DMA] * _NBUF,
        ),
    )(tokens, table)
